# baseline (speedup 1.0000x reference)
import functools

import jax
import jax.numpy as jnp
import numpy as np
from jax import lax
from jax.experimental import pallas as pl
from jax.experimental.pallas import tpu as pltpu
from jax.experimental.pallas import tpu_sc as plsc

W = 32
R_TOTAL = 16384
R_SC = 2048
R_TC = R_TOTAL - R_SC
SLAB = 64
NCHUNK = R_SC // SLAB // W
SEQ = 1024

_SCALE = 3.5
_STEP = 2.0 / 7.0

BR = 1024
N_TC_BLOCKS = R_TC // BR


def _codes_chunk(x_v, f_v):
    lane4 = lax.broadcasted_iota(jnp.int32, (16,), 0) * 4
    zeros16 = jnp.zeros((16,), jnp.int32)

    @plsc.parallel_loop(0, SLAB, 1, unroll=2)
    def row_blk(r):
        rows = zeros16 + r
        for qtr in range(4):
            ids = []
            for j in range(4):
                cols = lane4 + (qtr * 64 + j)
                x = plsc.load_gather(x_v, [rows, cols])
                t = x * _SCALE + 4.0
                t = jnp.minimum(jnp.maximum(t, 0.0), 7.5)
                ids.append(t.astype(jnp.int32))
            flat = ids[0] | (ids[1] << 3) | (ids[2] << 6) | (ids[3] << 9)
            f_v[r, pl.ds(qtr * 16, 16)] = flat


def _sc_body(x_hbm, f_hbm, x0, x1, f0, f1, si0, si1, so0, so1):
    wid = lax.axis_index("s") * 2 + lax.axis_index("c")
    xb, fb = [x0, x1], [f0, f1]
    si, so = [si0, si1], [so0, so1]
    in_copy = [None, None]
    out_f = [None, None]

    def x_slab(k):
        row = R_TC + k * SLAB
        b = row // SEQ
        r0 = pl.multiple_of(row % SEQ, SLAB)
        return x_hbm.at[b, pl.ds(r0, SLAB)]

    k0 = wid * NCHUNK
    in_copy[0] = pltpu.async_copy(x_slab(k0), xb[0], si[0])
    for c in range(NCHUNK):
        b = c & 1
        if c + 1 < NCHUNK:
            in_copy[1 - b] = pltpu.async_copy(
                x_slab(k0 + c + 1), xb[1 - b], si[1 - b])
        in_copy[b].wait()
        if c >= 2:
            out_f[b].wait()
        _codes_chunk(xb[b], fb[b])
        off = pl.multiple_of((k0 + c) * SLAB, SLAB)
        out_f[b] = pltpu.async_copy(fb[b], f_hbm.at[pl.ds(off, SLAB)], so[b])
    for b in range(min(2, NCHUNK)):
        out_f[b].wait()


@functools.partial(
    pl.kernel,
    out_type=jax.ShapeDtypeStruct((R_SC, 64), jnp.int32),
    mesh=plsc.VectorSubcoreMesh(core_axis_name="c", subcore_axis_name="s"),
    scratch_types=(
        [pltpu.VMEM((SLAB, 256), jnp.float32) for _ in range(2)]
        + [pltpu.VMEM((SLAB, 64), jnp.int32) for _ in range(2)]
        + [pltpu.SemaphoreType.DMA for _ in range(4)]
    ),
    compiler_params=pltpu.CompilerParams(needs_layout_passes=False),
)
def _sc_call(x_hbm, f_hbm, *bufs):
    _sc_body(x_hbm, f_hbm, *bufs)


def _tc_body(x_ref, s_ref, q_ref, f_ref):
    i = pl.program_id(0)
    x = x_ref[...]
    t = jnp.floor(jnp.clip(x * _SCALE + 4.0, 0.0, 7.5))
    q_ref[...] = t * _STEP - 1.0

    @pl.when(i < N_TC_BLOCKS)
    def _codes():
        f_ref[...] = jnp.dot(
            t.astype(jnp.bfloat16), s_ref[...],
            preferred_element_type=jnp.float32).astype(jnp.int32)


def _tc_call(x2, sel):
    return pl.pallas_call(
        _tc_body,
        grid=(R_TOTAL // BR,),
        in_specs=[
            pl.BlockSpec((BR, 256), lambda i: (i, 0)),
            pl.BlockSpec((256, 64), lambda i: (0, 0)),
        ],
        out_specs=[
            pl.BlockSpec((BR, 256), lambda i: (i, 0)),
            pl.BlockSpec((BR, 64), lambda i: (i, 0)),
        ],
        out_shape=[
            jax.ShapeDtypeStruct((R_TOTAL, 256), jnp.float32),
            jax.ShapeDtypeStruct((R_TOTAL, 64), jnp.int32),
        ],
    )(x2, sel)


_SEL = np.zeros((256, 64), dtype=np.float32)
for _d in range(256):
    _SEL[_d, _d // 4] = float((1, 8, 64, 512)[_d % 4])


@jax.jit
def kernel(latents):
    bsz, seq_len, dim = latents.shape
    sel = jnp.asarray(_SEL, dtype=jnp.bfloat16)
    f_sc = _sc_call(latents)
    q, f_tc = _tc_call(latents.reshape(R_TOTAL, 256), sel)
    f = lax.dynamic_update_slice(f_tc, f_sc, (R_TC, 0))
    return (
        q.reshape(bsz, seq_len, dim),
        f.reshape(bsz, seq_len, dim // 4),
    )


# speedup vs baseline: 1.1464x; 1.1464x over previous
import functools

import jax
import jax.numpy as jnp
import numpy as np
from jax import lax
from jax.experimental import pallas as pl
from jax.experimental.pallas import tpu as pltpu
from jax.experimental.pallas import tpu_sc as plsc

W = 32
R_TOTAL = 16384
R_SC = 2048
R_TC = R_TOTAL - R_SC
SLAB = 64
NCHUNK = R_SC // SLAB // W
SEQ = 1024

_SCALE = 3.5
_STEP = 2.0 / 7.0

BR = 4096
N_TC_BLOCKS = R_TC // BR


def _codes_chunk(x_v, f_v):
    lane4 = lax.broadcasted_iota(jnp.int32, (16,), 0) * 4
    zeros16 = jnp.zeros((16,), jnp.int32)

    @plsc.parallel_loop(0, SLAB, 1, unroll=2)
    def row_blk(r):
        rows = zeros16 + r
        for qtr in range(4):
            ids = []
            for j in range(4):
                cols = lane4 + (qtr * 64 + j)
                x = plsc.load_gather(x_v, [rows, cols])
                t = x * _SCALE + 4.0
                t = jnp.minimum(jnp.maximum(t, 0.0), 7.5)
                ids.append(t.astype(jnp.int32))
            flat = ids[0] | (ids[1] << 3) | (ids[2] << 6) | (ids[3] << 9)
            f_v[r, pl.ds(qtr * 16, 16)] = flat


def _sc_body(x_hbm, f_hbm, x0, x1, f0, f1, si0, si1, so0, so1):
    wid = lax.axis_index("s") * 2 + lax.axis_index("c")
    xb, fb = [x0, x1], [f0, f1]
    si, so = [si0, si1], [so0, so1]
    in_copy = [None, None]
    out_f = [None, None]

    def x_slab(k):
        row = R_TC + k * SLAB
        b = row // SEQ
        r0 = pl.multiple_of(row % SEQ, SLAB)
        return x_hbm.at[b, pl.ds(r0, SLAB)]

    k0 = wid * NCHUNK
    in_copy[0] = pltpu.async_copy(x_slab(k0), xb[0], si[0])
    for c in range(NCHUNK):
        b = c & 1
        if c + 1 < NCHUNK:
            in_copy[1 - b] = pltpu.async_copy(
                x_slab(k0 + c + 1), xb[1 - b], si[1 - b])
        in_copy[b].wait()
        if c >= 2:
            out_f[b].wait()
        _codes_chunk(xb[b], fb[b])
        off = pl.multiple_of((k0 + c) * SLAB, SLAB)
        out_f[b] = pltpu.async_copy(fb[b], f_hbm.at[pl.ds(off, SLAB)], so[b])
    for b in range(min(2, NCHUNK)):
        out_f[b].wait()


@functools.partial(
    pl.kernel,
    out_type=jax.ShapeDtypeStruct((R_SC, 64), jnp.int32),
    mesh=plsc.VectorSubcoreMesh(core_axis_name="c", subcore_axis_name="s"),
    scratch_types=(
        [pltpu.VMEM((SLAB, 256), jnp.float32) for _ in range(2)]
        + [pltpu.VMEM((SLAB, 64), jnp.int32) for _ in range(2)]
        + [pltpu.SemaphoreType.DMA for _ in range(4)]
    ),
    compiler_params=pltpu.CompilerParams(needs_layout_passes=False),
)
def _sc_call(x_hbm, f_hbm, *bufs):
    _sc_body(x_hbm, f_hbm, *bufs)


def _tc_body(x_ref, s_ref, q_ref, f_ref):
    i = pl.program_id(0)
    x = x_ref[...]
    t = jnp.floor(jnp.clip(x * _SCALE + 4.0, 0.0, 7.5))
    q_ref[...] = t * _STEP - 1.0

    @pl.when(i < N_TC_BLOCKS)
    def _codes():
        f_ref[...] = jnp.dot(
            t.astype(jnp.bfloat16), s_ref[...],
            preferred_element_type=jnp.float32).astype(jnp.int32)


def _tc_call(x2, sel):
    return pl.pallas_call(
        _tc_body,
        grid=(R_TOTAL // BR,),
        in_specs=[
            pl.BlockSpec((BR, 256), lambda i: (i, 0)),
            pl.BlockSpec((256, 64), lambda i: (0, 0)),
        ],
        out_specs=[
            pl.BlockSpec((BR, 256), lambda i: (i, 0)),
            pl.BlockSpec((BR, 64), lambda i: (i, 0)),
        ],
        out_shape=[
            jax.ShapeDtypeStruct((R_TOTAL, 256), jnp.float32),
            jax.ShapeDtypeStruct((R_TOTAL, 64), jnp.int32),
        ],
    )(x2, sel)


_SEL = np.zeros((256, 64), dtype=np.float32)
for _d in range(256):
    _SEL[_d, _d // 4] = float((1, 8, 64, 512)[_d % 4])


@jax.jit
def kernel(latents):
    bsz, seq_len, dim = latents.shape
    sel = jnp.asarray(_SEL, dtype=jnp.bfloat16)
    f_sc = _sc_call(latents)
    q, f_tc = _tc_call(latents.reshape(R_TOTAL, 256), sel)
    f = lax.dynamic_update_slice(f_tc, f_sc, (R_TC, 0))
    return (
        q.reshape(bsz, seq_len, dim),
        f.reshape(bsz, seq_len, dim // 4),
    )
